# baseline (speedup 1.0000x reference)
import functools

import jax
import jax.numpy as jnp
from jax import lax
from jax.experimental import pallas as pl
from jax.experimental.pallas import tpu as pltpu
from jax.experimental.pallas import tpu_sc as plsc

_SC_PARAMS = pltpu.CompilerParams(use_tc_tiling_on_sc=False)

NC = 2
NS = 16
NW = NC * NS
BATCH = 128
K = 10
D = 8
NPW = 128 // D
TCW = 784


def _make_deg(n_pad, r_pad):
    rows_per_worker = r_pad // NW
    rpt = n_pad // NS
    mesh = plsc.VectorSubcoreMesh(core_axis_name="c", subcore_axis_name="s")

    @functools.partial(
        pl.kernel,
        out_type=jax.ShapeDtypeStruct((NC, n_pad, D), jnp.float32),
        mesh=mesh,
        scratch_types=[
            pltpu.VMEM((2, K, BATCH), jnp.int32),
            pltpu.VMEM((BATCH, D), jnp.float32),
            pltpu.VMEM_SHARED((n_pad, D), jnp.float32),
            pltpu.SemaphoreType.DMA,
            pltpu.SemaphoreType.DMA,
        ],
        compiler_params=_SC_PARAMS,
    )
    def deg_kernel(ei_hbm, ones_hbm, zero_hbm, out_hbm,
                   dst_v, ones_v, deg_sh, sem_s0, sem_s1):
        cid = lax.axis_index("c")
        sid = lax.axis_index("s")
        w = cid * NS + sid
        r0 = sid * rpt
        pltpu.sync_copy(ones_hbm, ones_v)
        pltpu.sync_copy(zero_hbm.at[pl.ds(r0, rpt)], deg_sh.at[pl.ds(r0, rpt)])
        plsc.subcore_barrier()
        row0 = w * rows_per_worker
        sems = (sem_s0, sem_s1)

        def drain(b):
            for _ in range(K):
                pltpu.make_async_copy(
                    ones_v, deg_sh.at[pl.ds(0, BATCH)], sems[b]).wait()

        def chunk(r, b, t):
            @pl.when(t > 0)
            def _():
                drain(b)
            pltpu.sync_copy(ei_hbm.at[1, pl.ds(r, K)], dst_v.at[b])
            for j in range(K):
                pltpu.async_copy(ones_v, deg_sh.at[dst_v.at[b, j]],
                                 sems[b], add=True)

        @pl.loop(0, rows_per_worker // (2 * K))
        def _(t):
            chunk(row0 + t * 2 * K, 0, t)
            chunk(row0 + t * 2 * K + K, 1, t)

        drain(0)
        drain(1)
        plsc.subcore_barrier()
        pltpu.sync_copy(deg_sh.at[pl.ds(r0, rpt)],
                        out_hbm.at[cid, pl.ds(r0, rpt)])

    return deg_kernel


def _make_agg(n_pad, r_pad):
    rows_per_worker = r_pad // NW
    rpt = n_pad // NS
    mesh = plsc.VectorSubcoreMesh(core_axis_name="c", subcore_axis_name="s")

    @functools.partial(
        pl.kernel,
        out_type=jax.ShapeDtypeStruct((NC, n_pad, D), jnp.float32),
        mesh=mesh,
        scratch_types=[
            pltpu.VMEM((2, K, BATCH), jnp.int32),
            pltpu.VMEM((2, K, BATCH), jnp.int32),
            pltpu.VMEM((2, K * BATCH, D), jnp.float32),
            pltpu.VMEM_SHARED((n_pad, D), jnp.float32),
            pltpu.VMEM_SHARED((n_pad, D), jnp.float32),
            pltpu.SemaphoreType.DMA,
            pltpu.SemaphoreType.DMA,
            pltpu.SemaphoreType.DMA,
        ],
        compiler_params=_SC_PARAMS,
    )
    def agg_kernel(table_hbm, ei_hbm, zero_hbm, out_hbm,
                   src_v, dst_v, rows_v, table_sh, acc_sh,
                   sem_g, sem_s0, sem_s1):
        cid = lax.axis_index("c")
        sid = lax.axis_index("s")
        w = cid * NS + sid
        r0 = sid * rpt
        pltpu.sync_copy(table_hbm.at[pl.ds(r0, rpt)], table_sh.at[pl.ds(r0, rpt)])
        pltpu.sync_copy(zero_hbm.at[pl.ds(r0, rpt)], acc_sh.at[pl.ds(r0, rpt)])
        plsc.subcore_barrier()
        row0 = w * rows_per_worker
        sems = (sem_s0, sem_s1)

        def drain(b):
            for _ in range(K):
                pltpu.make_async_copy(
                    rows_v.at[b, pl.ds(0, BATCH)],
                    acc_sh.at[pl.ds(0, BATCH)], sems[b]).wait()

        def chunk(r, b, t):
            @pl.when(t > 0)
            def _():
                drain(b)
            pltpu.sync_copy(ei_hbm.at[0, pl.ds(r, K)], src_v.at[b])
            pltpu.sync_copy(ei_hbm.at[1, pl.ds(r, K)], dst_v.at[b])
            gathers = [
                pltpu.async_copy(table_sh.at[src_v.at[b, j]],
                                 rows_v.at[b, pl.ds(j * BATCH, BATCH)], sem_g)
                for j in range(K)
            ]
            for j in range(K):
                gathers[j].wait()
                pltpu.async_copy(rows_v.at[b, pl.ds(j * BATCH, BATCH)],
                                 acc_sh.at[dst_v.at[b, j]], sems[b], add=True)

        @pl.loop(0, rows_per_worker // (2 * K))
        def _(t):
            chunk(row0 + t * 2 * K, 0, t)
            chunk(row0 + t * 2 * K + K, 1, t)

        drain(0)
        drain(1)
        plsc.subcore_barrier()
        pltpu.sync_copy(acc_sh.at[pl.ds(r0, rpt)],
                        out_hbm.at[cid, pl.ds(r0, rpt)])

    return agg_kernel


def _d1(degw, x8w):
    w_rows = x8w.shape[0]

    def body(deg_ref, x_ref, dinv_ref, xs_ref):
        dinv = lax.rsqrt(deg_ref[0] + deg_ref[1] + 1.0)
        dinv_ref[...] = dinv
        xs_ref[...] = x_ref[...] * dinv

    return pl.pallas_call(
        body,
        grid=(w_rows // TCW,),
        in_specs=[
            pl.BlockSpec((NC, TCW, 128), lambda i: (0, i, 0)),
            pl.BlockSpec((TCW, 128), lambda i: (i, 0)),
        ],
        out_specs=[
            pl.BlockSpec((TCW, 128), lambda i: (i, 0)),
            pl.BlockSpec((TCW, 128), lambda i: (i, 0)),
        ],
        out_shape=[
            jax.ShapeDtypeStruct((w_rows, 128), jnp.float32),
            jax.ShapeDtypeStruct((w_rows, 128), jnp.float32),
        ],
    )(degw, x8w)


def _d2(acc1w, xs8w, dinvw, g1, b1t, g2):
    w_rows = xs8w.shape[0]
    dk = g1.shape[1]

    def body(acc_ref, xs_ref, dinv_ref, g1_ref, b1_ref, g2_ref, zs_ref):
        dinv = dinv_ref[...]
        a = (acc_ref[0] + acc_ref[1] + xs_ref[...]) * dinv
        h = jnp.dot(a, g1_ref[...], preferred_element_type=jnp.float32)
        h = jnp.maximum(h + b1_ref[...], 0.0)
        z = jnp.dot(h, g2_ref[...], preferred_element_type=jnp.float32)
        zs_ref[...] = z * dinv

    return pl.pallas_call(
        body,
        grid=(w_rows // TCW,),
        in_specs=[
            pl.BlockSpec((NC, TCW, 128), lambda i: (0, i, 0)),
            pl.BlockSpec((TCW, 128), lambda i: (i, 0)),
            pl.BlockSpec((TCW, 128), lambda i: (i, 0)),
            pl.BlockSpec((128, dk), lambda i: (0, 0)),
            pl.BlockSpec((1, dk), lambda i: (0, 0)),
            pl.BlockSpec((dk, 128), lambda i: (0, 0)),
        ],
        out_specs=pl.BlockSpec((TCW, 128), lambda i: (i, 0)),
        out_shape=jax.ShapeDtypeStruct((w_rows, 128), jnp.float32),
    )(acc1w, xs8w, dinvw, g1, b1t, g2)


def _d3(acc2w, zsw, dinvw, b2t, x8w, xmask):
    w_rows = zsw.shape[0]

    def body(acc_ref, zs_ref, dinv_ref, b2_ref, x_ref, m_ref, h2_ref):
        h2_ref[...] = (((acc_ref[0] + acc_ref[1] + zs_ref[...])
                        * dinv_ref[...]) + b2_ref[...]
                       + x_ref[...] * m_ref[...])

    return pl.pallas_call(
        body,
        grid=(w_rows // TCW,),
        in_specs=[
            pl.BlockSpec((NC, TCW, 128), lambda i: (0, i, 0)),
            pl.BlockSpec((TCW, 128), lambda i: (i, 0)),
            pl.BlockSpec((TCW, 128), lambda i: (i, 0)),
            pl.BlockSpec((1, 128), lambda i: (0, 0)),
            pl.BlockSpec((TCW, 128), lambda i: (i, 0)),
            pl.BlockSpec((1, 128), lambda i: (0, 0)),
        ],
        out_specs=pl.BlockSpec((TCW, 128), lambda i: (i, 0)),
        out_shape=jax.ShapeDtypeStruct((w_rows, 128), jnp.float32),
    )(acc2w, zsw, dinvw, b2t, x8w, xmask)


def kernel(x, edge_index, W1, b1, W2, b2):
    N, din = x.shape
    dh = W1.shape[1]
    dout = W2.shape[1]
    E = edge_index.shape[1]

    n_pad = -(-N // (TCW * NPW)) * (TCW * NPW)
    w_rows = n_pad // NPW

    e128 = -(-E // BATCH) * BATCH
    ei = edge_index
    if e128 != E:
        ei = jnp.pad(ei, ((0, 0), (0, e128 - E)), constant_values=N)
    r_rows = e128 // BATCH
    r_pad = -(-r_rows // (NW * K * 2)) * (NW * K * 2)
    ei3 = ei.reshape(2, r_rows, BATCH)
    if r_pad != r_rows:
        ei3 = jnp.pad(ei3, ((0, 0), (0, r_pad - r_rows), (0, 0)),
                      constant_values=N)

    x8w = (jnp.zeros((n_pad, D), jnp.float32).at[:N, :din].set(x)
           .reshape(w_rows, 128))

    eye16 = jnp.eye(NPW, dtype=jnp.float32)
    w1p = jnp.zeros((D, dh), jnp.float32).at[:din, :].set(W1)
    w2p = jnp.zeros((dh, D), jnp.float32).at[:, 3:3 + dout].set(W2)
    g1 = jnp.kron(eye16, w1p)
    g2 = jnp.kron(eye16, w2p)
    b1t = jnp.tile(b1, NPW).reshape(1, NPW * dh)
    b2p = jnp.zeros((D,), jnp.float32).at[3:3 + dout].set(b2)
    b2t = jnp.tile(b2p, NPW).reshape(1, 128)
    xm = jnp.zeros((D,), jnp.float32).at[:3].set(1.0)
    xmask = jnp.tile(xm, NPW).reshape(1, 128)

    ones8 = jnp.ones((BATCH, D), jnp.float32)
    zero8 = jnp.zeros((n_pad, D), jnp.float32)

    def widen(a):
        return a.reshape(NC, w_rows, 128)

    deg = _make_deg(n_pad, r_pad)(ei3, ones8, zero8)

    dinvw, xs8w = _d1(widen(deg), x8w)

    acc1 = _make_agg(n_pad, r_pad)(xs8w.reshape(n_pad, D), ei3, zero8)

    zsw = _d2(widen(acc1), xs8w, dinvw, g1, b1t, g2)

    acc2 = _make_agg(n_pad, r_pad)(zsw.reshape(n_pad, D), ei3, zero8)

    outw = _d3(widen(acc2), zsw, dinvw, b2t, x8w, xmask)

    if (N * D) % 128 == 0:
        return outw[:(N * D) // 128].reshape(N, D)[:, :3 + dout]
    return outw.reshape(n_pad, D)[:N, :3 + dout]


# speedup vs baseline: 1.0135x; 1.0135x over previous
import functools

import jax
import jax.numpy as jnp
from jax import lax
from jax.experimental import pallas as pl
from jax.experimental.pallas import tpu as pltpu
from jax.experimental.pallas import tpu_sc as plsc

_SC_PARAMS = pltpu.CompilerParams(use_tc_tiling_on_sc=False)

NC = 2
NS = 16
NW = NC * NS
BATCH = 128
K = 10
D = 8
NPW = 128 // D
TCW = 784


def _make_deg(n_pad, r_pad):
    rows_per_worker = r_pad // NW
    rpt = n_pad // NS
    mesh = plsc.VectorSubcoreMesh(core_axis_name="c", subcore_axis_name="s")

    @functools.partial(
        pl.kernel,
        out_type=jax.ShapeDtypeStruct((NC, n_pad, D), jnp.float32),
        mesh=mesh,
        scratch_types=[
            pltpu.VMEM((2, K, BATCH), jnp.int32),
            pltpu.VMEM((BATCH, D), jnp.float32),
            pltpu.VMEM_SHARED((n_pad, D), jnp.float32),
            pltpu.SemaphoreType.DMA,
            pltpu.SemaphoreType.DMA,
        ],
        compiler_params=_SC_PARAMS,
    )
    def deg_kernel(ei_hbm, ones_hbm, zero_hbm, out_hbm,
                   dst_v, ones_v, deg_sh, sem_s0, sem_s1):
        cid = lax.axis_index("c")
        sid = lax.axis_index("s")
        w = cid * NS + sid
        r0 = sid * rpt
        pltpu.sync_copy(ones_hbm, ones_v)
        pltpu.sync_copy(zero_hbm.at[pl.ds(r0, rpt)], deg_sh.at[pl.ds(r0, rpt)])
        plsc.subcore_barrier()
        row0 = w * rows_per_worker
        sems = (sem_s0, sem_s1)

        def drain(b):
            for _ in range(K):
                pltpu.make_async_copy(
                    ones_v, deg_sh.at[pl.ds(0, BATCH)], sems[b]).wait()

        def chunk(r, b, t):
            @pl.when(t > 0)
            def _():
                drain(b)
            pltpu.sync_copy(ei_hbm.at[1, pl.ds(r, K)], dst_v.at[b])
            for j in range(K):
                pltpu.async_copy(ones_v, deg_sh.at[dst_v.at[b, j]],
                                 sems[b], add=True)

        @pl.loop(0, rows_per_worker // (2 * K))
        def _(t):
            chunk(row0 + t * 2 * K, 0, t)
            chunk(row0 + t * 2 * K + K, 1, t)

        drain(0)
        drain(1)
        plsc.subcore_barrier()
        pltpu.sync_copy(deg_sh.at[pl.ds(r0, rpt)],
                        out_hbm.at[cid, pl.ds(r0, rpt)])

    return deg_kernel


def _make_agg(n_pad, r_pad):
    rows_per_worker = r_pad // NW
    rpt = n_pad // NS
    mesh = plsc.VectorSubcoreMesh(core_axis_name="c", subcore_axis_name="s")

    @functools.partial(
        pl.kernel,
        out_type=jax.ShapeDtypeStruct((NC, n_pad, D), jnp.float32),
        mesh=mesh,
        scratch_types=[
            pltpu.VMEM((2, K, BATCH), jnp.int32),
            pltpu.VMEM((2, K, BATCH), jnp.int32),
            pltpu.VMEM((2, K * BATCH, D), jnp.float32),
            pltpu.VMEM_SHARED((n_pad, D), jnp.float32),
            pltpu.VMEM_SHARED((n_pad, D), jnp.float32),
            pltpu.SemaphoreType.DMA,
            pltpu.SemaphoreType.DMA,
            pltpu.SemaphoreType.DMA,
        ],
        compiler_params=_SC_PARAMS,
    )
    def agg_kernel(table_hbm, ei_hbm, zero_hbm, out_hbm,
                   src_v, dst_v, rows_v, table_sh, acc_sh,
                   sem_g, sem_s0, sem_s1):
        cid = lax.axis_index("c")
        sid = lax.axis_index("s")
        w = cid * NS + sid
        r0 = sid * rpt
        pltpu.sync_copy(table_hbm.at[pl.ds(r0, rpt)], table_sh.at[pl.ds(r0, rpt)])
        pltpu.sync_copy(zero_hbm.at[pl.ds(r0, rpt)], acc_sh.at[pl.ds(r0, rpt)])
        plsc.subcore_barrier()
        row0 = w * rows_per_worker
        sems = (sem_s0, sem_s1)

        def drain(b):
            for _ in range(K):
                pltpu.make_async_copy(
                    rows_v.at[b, pl.ds(0, BATCH)],
                    acc_sh.at[pl.ds(0, BATCH)], sems[b]).wait()

        def chunk(r, b, t):
            @pl.when(t > 0)
            def _():
                drain(b)
            pltpu.sync_copy(ei_hbm.at[0, pl.ds(r, K)], src_v.at[b])
            pltpu.sync_copy(ei_hbm.at[1, pl.ds(r, K)], dst_v.at[b])
            gathers = [
                pltpu.async_copy(table_sh.at[src_v.at[b, j]],
                                 rows_v.at[b, pl.ds(j * BATCH, BATCH)], sem_g)
                for j in range(K)
            ]
            for g in gathers:
                g.wait()
            for j in range(K):
                pltpu.async_copy(rows_v.at[b, pl.ds(j * BATCH, BATCH)],
                                 acc_sh.at[dst_v.at[b, j]], sems[b], add=True)

        @pl.loop(0, rows_per_worker // (2 * K))
        def _(t):
            chunk(row0 + t * 2 * K, 0, t)
            chunk(row0 + t * 2 * K + K, 1, t)

        drain(0)
        drain(1)
        plsc.subcore_barrier()
        pltpu.sync_copy(acc_sh.at[pl.ds(r0, rpt)],
                        out_hbm.at[cid, pl.ds(r0, rpt)])

    return agg_kernel


def _d1(degw, x8w):
    w_rows = x8w.shape[0]

    def body(deg_ref, x_ref, dinv_ref, xs_ref):
        dinv = lax.rsqrt(deg_ref[0] + deg_ref[1] + 1.0)
        dinv_ref[...] = dinv
        xs_ref[...] = x_ref[...] * dinv

    return pl.pallas_call(
        body,
        grid=(w_rows // TCW,),
        in_specs=[
            pl.BlockSpec((NC, TCW, 128), lambda i: (0, i, 0)),
            pl.BlockSpec((TCW, 128), lambda i: (i, 0)),
        ],
        out_specs=[
            pl.BlockSpec((TCW, 128), lambda i: (i, 0)),
            pl.BlockSpec((TCW, 128), lambda i: (i, 0)),
        ],
        out_shape=[
            jax.ShapeDtypeStruct((w_rows, 128), jnp.float32),
            jax.ShapeDtypeStruct((w_rows, 128), jnp.float32),
        ],
    )(degw, x8w)


def _d2(acc1w, xs8w, dinvw, g1, b1t, g2):
    w_rows = xs8w.shape[0]
    dk = g1.shape[1]

    def body(acc_ref, xs_ref, dinv_ref, g1_ref, b1_ref, g2_ref, zs_ref):
        dinv = dinv_ref[...]
        a = (acc_ref[0] + acc_ref[1] + xs_ref[...]) * dinv
        h = jnp.dot(a, g1_ref[...], preferred_element_type=jnp.float32)
        h = jnp.maximum(h + b1_ref[...], 0.0)
        z = jnp.dot(h, g2_ref[...], preferred_element_type=jnp.float32)
        zs_ref[...] = z * dinv

    return pl.pallas_call(
        body,
        grid=(w_rows // TCW,),
        in_specs=[
            pl.BlockSpec((NC, TCW, 128), lambda i: (0, i, 0)),
            pl.BlockSpec((TCW, 128), lambda i: (i, 0)),
            pl.BlockSpec((TCW, 128), lambda i: (i, 0)),
            pl.BlockSpec((128, dk), lambda i: (0, 0)),
            pl.BlockSpec((1, dk), lambda i: (0, 0)),
            pl.BlockSpec((dk, 128), lambda i: (0, 0)),
        ],
        out_specs=pl.BlockSpec((TCW, 128), lambda i: (i, 0)),
        out_shape=jax.ShapeDtypeStruct((w_rows, 128), jnp.float32),
    )(acc1w, xs8w, dinvw, g1, b1t, g2)


def _d3(acc2w, zsw, dinvw, b2t, x8w, xmask):
    w_rows = zsw.shape[0]

    def body(acc_ref, zs_ref, dinv_ref, b2_ref, x_ref, m_ref, h2_ref):
        h2_ref[...] = (((acc_ref[0] + acc_ref[1] + zs_ref[...])
                        * dinv_ref[...]) + b2_ref[...]
                       + x_ref[...] * m_ref[...])

    return pl.pallas_call(
        body,
        grid=(w_rows // TCW,),
        in_specs=[
            pl.BlockSpec((NC, TCW, 128), lambda i: (0, i, 0)),
            pl.BlockSpec((TCW, 128), lambda i: (i, 0)),
            pl.BlockSpec((TCW, 128), lambda i: (i, 0)),
            pl.BlockSpec((1, 128), lambda i: (0, 0)),
            pl.BlockSpec((TCW, 128), lambda i: (i, 0)),
            pl.BlockSpec((1, 128), lambda i: (0, 0)),
        ],
        out_specs=pl.BlockSpec((TCW, 128), lambda i: (i, 0)),
        out_shape=jax.ShapeDtypeStruct((w_rows, 128), jnp.float32),
    )(acc2w, zsw, dinvw, b2t, x8w, xmask)


def kernel(x, edge_index, W1, b1, W2, b2):
    N, din = x.shape
    dh = W1.shape[1]
    dout = W2.shape[1]
    E = edge_index.shape[1]

    n_pad = -(-N // (TCW * NPW)) * (TCW * NPW)
    w_rows = n_pad // NPW

    e128 = -(-E // BATCH) * BATCH
    ei = edge_index
    if e128 != E:
        ei = jnp.pad(ei, ((0, 0), (0, e128 - E)), constant_values=N)
    r_rows = e128 // BATCH
    r_pad = -(-r_rows // (NW * K * 2)) * (NW * K * 2)
    ei3 = ei.reshape(2, r_rows, BATCH)
    if r_pad != r_rows:
        ei3 = jnp.pad(ei3, ((0, 0), (0, r_pad - r_rows), (0, 0)),
                      constant_values=N)

    x8w = (jnp.zeros((n_pad, D), jnp.float32).at[:N, :din].set(x)
           .reshape(w_rows, 128))

    eye16 = jnp.eye(NPW, dtype=jnp.float32)
    w1p = jnp.zeros((D, dh), jnp.float32).at[:din, :].set(W1)
    w2p = jnp.zeros((dh, D), jnp.float32).at[:, 3:3 + dout].set(W2)
    g1 = jnp.kron(eye16, w1p)
    g2 = jnp.kron(eye16, w2p)
    b1t = jnp.tile(b1, NPW).reshape(1, NPW * dh)
    b2p = jnp.zeros((D,), jnp.float32).at[3:3 + dout].set(b2)
    b2t = jnp.tile(b2p, NPW).reshape(1, 128)
    xm = jnp.zeros((D,), jnp.float32).at[:3].set(1.0)
    xmask = jnp.tile(xm, NPW).reshape(1, 128)

    ones8 = jnp.ones((BATCH, D), jnp.float32)
    zero8 = jnp.zeros((n_pad, D), jnp.float32)

    def widen(a):
        return a.reshape(NC, w_rows, 128)

    deg = _make_deg(n_pad, r_pad)(ei3, ones8, zero8)

    dinvw, xs8w = _d1(widen(deg), x8w)

    acc1 = _make_agg(n_pad, r_pad)(xs8w.reshape(n_pad, D), ei3, zero8)

    zsw = _d2(widen(acc1), xs8w, dinvw, g1, b1t, g2)

    acc2 = _make_agg(n_pad, r_pad)(zsw.reshape(n_pad, D), ei3, zero8)

    outw = _d3(widen(acc2), zsw, dinvw, b2t, x8w, xmask)

    if (N * D) % 128 == 0:
        return outw[:(N * D) // 128].reshape(N, D)[:, :3 + dout]
    return outw.reshape(n_pad, D)[:N, :3 + dout]
